# baseline (speedup 1.0000x reference)
import functools

import jax
import jax.numpy as jnp
from jax import lax
from jax.experimental import pallas as pl
from jax.experimental.pallas import tpu as pltpu
from jax.experimental.pallas import tpu_sc as plsc

_IN_DIM = 4096
_N_EXPERTS = 64
_TOP_K = 8
_N_TOKENS = 16384
_ROWS = 1024

_NCHUNK = 2
_CROWS = _N_TOKENS // _NCHUNK

_NC = 2
_NS = 16
_NW = _NC * _NS
_RPW = _CROWS // _NW
_CHUNK = 256


def _cv2(v):
    n = v.shape[-1]
    mean = jnp.sum(v) / n
    var = jnp.sum((v - mean) ** 2) / (n - 1)
    return var / (mean * mean + 1e-10)


def _matmul_body(x_ref, w_ref, logits_ref, zsum_ref):
    i = pl.program_id(0)

    @pl.when(i == 0)
    def _init():
        zsum_ref[0, 0] = jnp.float32(0.0)

    logits = jnp.dot(x_ref[:], w_ref[:], preferred_element_type=jnp.float32)
    logits_ref[:] = logits
    rowmax = jnp.max(logits, axis=1, keepdims=True)
    lse = rowmax[:, 0] + jnp.log(jnp.sum(jnp.exp(logits - rowmax), axis=1))
    zsum_ref[0, 0] += jnp.sum(lse)


def _lane_perms():
    iota = lax.iota(jnp.int32, 16)
    return [jnp.reshape(iota ^ d, (16, 1)) for d in (1, 2, 4, 8)]


_GATHER_DNUMS = lax.GatherDimensionNumbers(
    offset_dims=(), collapsed_slice_dims=(0,), start_index_map=(0,))


def _shuffle(x, p):
    return lax.gather(x, p, dimension_numbers=_GATHER_DNUMS,
                      slice_sizes=(1,),
                      mode=lax.GatherScatterMode.PROMISE_IN_BOUNDS)


def _splat_reduce(x, op, perms):
    for p in perms:
        x = op(x, _shuffle(x, p))
    return x


def _route_body(logits_hbm, gates_hbm, imp_hbm, cnt_hbm,
                in_v, out_v, stat_v, cntstat_v):
    wid = lax.axis_index("s") * _NC + lax.axis_index("c")
    base = wid * _RPW

    neg = jnp.float32(-jnp.inf)
    zf = jnp.float32(0.0)
    perms = _lane_perms()

    def row(r, carry):
        accs = list(carry)
        v = [in_v[r, pl.ds(16 * k, 16)] for k in range(4)]
        w = list(v)
        m = _splat_reduce(
            jnp.maximum(jnp.maximum(w[0], w[1]), jnp.maximum(w[2], w[3])),
            jnp.maximum, perms)
        rowmax = m
        for _ in range(_TOP_K):
            for k in range(4):
                w[k] = jnp.where(w[k] == m, neg, w[k])
            m = _splat_reduce(
                jnp.maximum(jnp.maximum(w[0], w[1]), jnp.maximum(w[2], w[3])),
                jnp.maximum, perms)
        g = [jnp.where(w[k] != v[k], jnp.exp(v[k] - rowmax), zf)
             for k in range(4)]
        denom = _splat_reduce(g[0] + g[1] + g[2] + g[3], jnp.add, perms)
        inv = jnp.float32(1.0) / denom
        out = []
        for k in range(4):
            gk = g[k] * inv
            out_v[r, pl.ds(16 * k, 16)] = gk
            out.append(gk)
        new = []
        for k in range(4):
            new.append(accs[k] + out[k])
        for k in range(4):
            new.append(accs[4 + k]
                       + jnp.where(out[k] > zf, jnp.int32(1), jnp.int32(0)))
        return tuple(new)

    zero_f = jnp.zeros((16,), jnp.float32)
    zero_i = jnp.zeros((16,), jnp.int32)
    carry = (zero_f,) * 4 + (zero_i,) * 4
    for c in range(_RPW // _CHUNK):
        pltpu.sync_copy(logits_hbm.at[pl.ds(base + c * _CHUNK, _CHUNK)], in_v)
        carry = lax.fori_loop(0, _CHUNK, row, carry)
        pltpu.sync_copy(out_v, gates_hbm.at[pl.ds(base + c * _CHUNK, _CHUNK)])

    for k in range(4):
        stat_v[pl.ds(16 * k, 16)] = carry[k]
        cntstat_v[pl.ds(16 * k, 16)] = carry[4 + k]
    pltpu.sync_copy(stat_v, imp_hbm.at[wid])
    pltpu.sync_copy(cntstat_v, cnt_hbm.at[wid])


def _epilogue_body(imp0_ref, imp1_ref, cnt0_ref, cnt1_ref, z0_ref, z1_ref,
                   importance_ref, load_ref, loss_ref):
    imp = (jnp.sum(imp0_ref[:], axis=0, keepdims=True)
           + jnp.sum(imp1_ref[:], axis=0, keepdims=True))
    load = (jnp.sum(cnt0_ref[:], axis=0, keepdims=True)
            + jnp.sum(cnt1_ref[:], axis=0, keepdims=True))
    importance_ref[:] = imp
    load_ref[:] = load
    z = (z0_ref[0, 0] + z1_ref[0, 0]) / jnp.float32(_N_TOKENS)
    loss_ref[0, 0] = (_cv2(imp.reshape(_N_EXPERTS))
                      + _cv2(load.astype(jnp.float32).reshape(_N_EXPERTS))
                      + z)


@jax.jit
def kernel(flat_tokens, gate_weight, noise_weight):
    del noise_weight

    mesh = plsc.VectorSubcoreMesh(core_axis_name="c", subcore_axis_name="s")
    route = functools.partial(
        pl.kernel,
        mesh=mesh,
        out_type=(
            jax.ShapeDtypeStruct((_CROWS, _N_EXPERTS), jnp.float32),
            jax.ShapeDtypeStruct((_NW, _N_EXPERTS), jnp.float32),
            jax.ShapeDtypeStruct((_NW, _N_EXPERTS), jnp.int32),
        ),
        scratch_types=[
            pltpu.VMEM((_CHUNK, _N_EXPERTS), jnp.float32),
            pltpu.VMEM((_CHUNK, _N_EXPERTS), jnp.float32),
            pltpu.VMEM((_N_EXPERTS,), jnp.float32),
            pltpu.VMEM((_N_EXPERTS,), jnp.int32),
        ],
    )(_route_body)

    logits_c, zsum_c, gates_c, imp_c, cnt_c = [], [], [], [], []
    steps = _CROWS // _ROWS
    for ch in range(_NCHUNK):
        logits, zsum = pl.pallas_call(
            _matmul_body,
            grid=(steps,),
            in_specs=[
                pl.BlockSpec((_ROWS, _IN_DIM),
                             lambda i, _c=ch: (_c * steps + i, 0)),
                pl.BlockSpec((_IN_DIM, _N_EXPERTS), lambda i: (0, 0)),
            ],
            out_specs=(
                pl.BlockSpec((_ROWS, _N_EXPERTS), lambda i: (i, 0)),
                pl.BlockSpec(memory_space=pltpu.SMEM),
            ),
            out_shape=(
                jax.ShapeDtypeStruct((_CROWS, _N_EXPERTS), jnp.float32),
                jax.ShapeDtypeStruct((1, 1), jnp.float32),
            ),
        )(flat_tokens, gate_weight)
        logits_c.append(logits)
        zsum_c.append(zsum)
        gates, imp, cnt = route(logits)
        gates_c.append(gates)
        imp_c.append(imp)
        cnt_c.append(cnt)

    importance, load, loss = pl.pallas_call(
        _epilogue_body,
        in_specs=[
            pl.BlockSpec((_NW, _N_EXPERTS), lambda: (0, 0)),
            pl.BlockSpec((_NW, _N_EXPERTS), lambda: (0, 0)),
            pl.BlockSpec((_NW, _N_EXPERTS), lambda: (0, 0)),
            pl.BlockSpec((_NW, _N_EXPERTS), lambda: (0, 0)),
            pl.BlockSpec(memory_space=pltpu.SMEM),
            pl.BlockSpec(memory_space=pltpu.SMEM),
        ],
        out_specs=(
            pl.BlockSpec((1, _N_EXPERTS), lambda: (0, 0)),
            pl.BlockSpec((1, _N_EXPERTS), lambda: (0, 0)),
            pl.BlockSpec(memory_space=pltpu.SMEM),
        ),
        out_shape=(
            jax.ShapeDtypeStruct((1, _N_EXPERTS), jnp.float32),
            jax.ShapeDtypeStruct((1, _N_EXPERTS), jnp.int32),
            jax.ShapeDtypeStruct((1, 1), jnp.float32),
        ),
    )(imp_c[0], imp_c[1], cnt_c[0], cnt_c[1], zsum_c[0], zsum_c[1])

    gates_full = jnp.concatenate(gates_c, axis=0)
    logits_full = jnp.concatenate(logits_c, axis=0)
    return (gates_full, load.reshape(_N_EXPERTS), logits_full, loss[0, 0],
            importance.reshape(_N_EXPERTS))


# speedup vs baseline: 1.2002x; 1.2002x over previous
import functools

import jax
import jax.numpy as jnp
from jax.experimental import pallas as pl
from jax.experimental.pallas import tpu as pltpu

_IN_DIM = 4096
_N_EXPERTS = 64
_TOP_K = 8
_N_TOKENS = 16384
_ROWS = 1024


def _cv2(v):
    n = v.shape[-1]
    mean = jnp.sum(v) / n
    var = jnp.sum((v - mean) ** 2) / (n - 1)
    return var / (mean * mean + 1e-10)


def _router_body(x_ref, w_ref, logits_ref, gates_ref, imp_ref, load_ref,
                 loss_ref, zsum_ref):
    i = pl.program_id(0)
    nsteps = pl.num_programs(0)

    @pl.when(i == 0)
    def _init():
        imp_ref[:] = jnp.zeros_like(imp_ref)
        load_ref[:] = jnp.zeros_like(load_ref)
        zsum_ref[0, 0] = jnp.float32(0.0)

    logits = jnp.dot(x_ref[:], w_ref[:], preferred_element_type=jnp.float32)
    logits_ref[:] = logits

    neg = jnp.float32(-jnp.inf)
    lane = jax.lax.broadcasted_iota(jnp.int32, logits.shape, 1)
    work = logits
    rowmax = jnp.max(work, axis=1, keepdims=True)
    m = rowmax
    for _ in range(_TOP_K):
        eq = work == m
        idx = jnp.min(jnp.where(eq, lane, _N_EXPERTS), axis=1, keepdims=True)
        work = jnp.where(lane == idx, neg, work)
        m = jnp.max(work, axis=1, keepdims=True)

    e_all = jnp.exp(logits - rowmax)
    e = jnp.where(work == logits, jnp.float32(0.0), e_all)
    denom = jnp.sum(e, axis=1, keepdims=True)
    gates = e / denom
    gates_ref[:] = gates

    imp_ref[:] += jnp.sum(gates, axis=0, keepdims=True)
    load_ref[:] += jnp.sum((gates > 0).astype(jnp.int32), axis=0,
                           keepdims=True)
    lse = rowmax[:, 0] + jnp.log(jnp.sum(e_all, axis=1))
    zsum_ref[0, 0] += jnp.sum(lse)

    @pl.when(i == nsteps - 1)
    def _finish():
        imp = imp_ref[:].reshape(_N_EXPERTS)
        load = load_ref[:].astype(jnp.float32).reshape(_N_EXPERTS)
        z = zsum_ref[0, 0] / jnp.float32(_N_TOKENS)
        loss_ref[0, 0] = _cv2(imp) + _cv2(load) + z


@jax.jit
def kernel(flat_tokens, gate_weight, noise_weight):
    del noise_weight
    n_tokens = flat_tokens.shape[0]
    grid = (n_tokens // _ROWS,)
    out_shape = (
        jax.ShapeDtypeStruct((n_tokens, _N_EXPERTS), jnp.float32),
        jax.ShapeDtypeStruct((n_tokens, _N_EXPERTS), jnp.float32),
        jax.ShapeDtypeStruct((1, _N_EXPERTS), jnp.float32),
        jax.ShapeDtypeStruct((1, _N_EXPERTS), jnp.int32),
        jax.ShapeDtypeStruct((1, 1), jnp.float32),
    )
    in_specs = [
        pl.BlockSpec((_ROWS, _IN_DIM), lambda i: (i, 0)),
        pl.BlockSpec((_IN_DIM, _N_EXPERTS), lambda i: (0, 0)),
    ]
    out_specs = (
        pl.BlockSpec((_ROWS, _N_EXPERTS), lambda i: (i, 0)),
        pl.BlockSpec((_ROWS, _N_EXPERTS), lambda i: (i, 0)),
        pl.BlockSpec((1, _N_EXPERTS), lambda i: (0, 0)),
        pl.BlockSpec((1, _N_EXPERTS), lambda i: (0, 0)),
        pl.BlockSpec(memory_space=pltpu.SMEM),
    )
    logits, gates, imp, load, loss = pl.pallas_call(
        _router_body,
        grid=grid,
        in_specs=in_specs,
        out_specs=out_specs,
        out_shape=out_shape,
        scratch_shapes=[pltpu.SMEM((1, 1), jnp.float32)],
    )(flat_tokens, gate_weight)
    return (gates, load.reshape(_N_EXPERTS), logits, loss[0, 0],
            imp.reshape(_N_EXPERTS))


# speedup vs baseline: 1.3837x; 1.1529x over previous
import functools

import jax
import jax.numpy as jnp
from jax.experimental import pallas as pl
from jax.experimental.pallas import tpu as pltpu

_IN_DIM = 4096
_N_EXPERTS = 64
_TOP_K = 8
_N_TOKENS = 16384
_ROWS = 1024


def _cv2(v):
    n = v.shape[-1]
    mean = jnp.sum(v) / n
    var = jnp.sum((v - mean) ** 2) / (n - 1)
    return var / (mean * mean + 1e-10)


def _router_body(x_ref, w_ref, logits_ref, gates_ref, imp_ref, load_ref,
                 loss_ref, zsum_ref):
    i = pl.program_id(0)
    nsteps = pl.num_programs(0)

    @pl.when(i == 0)
    def _init():
        imp_ref[:] = jnp.zeros_like(imp_ref)
        load_ref[:] = jnp.zeros_like(load_ref)
        zsum_ref[0, 0] = jnp.float32(0.0)

    logits = jnp.dot(x_ref[:], w_ref[:], preferred_element_type=jnp.float32)
    logits_ref[:] = logits

    neg = jnp.float32(-jnp.inf)
    work = logits
    rowmax = jnp.max(work, axis=1, keepdims=True)
    m = rowmax
    for _ in range(_TOP_K):
        work = jnp.where(work == m, neg, work)
        m = jnp.max(work, axis=1, keepdims=True)

    e_all = jnp.exp(logits - rowmax)
    e = jnp.where(work == logits, jnp.float32(0.0), e_all)
    denom = jnp.sum(e, axis=1, keepdims=True)
    gates = e / denom
    gates_ref[:] = gates

    imp_ref[:] += jnp.sum(gates, axis=0, keepdims=True)
    load_ref[:] += jnp.sum((gates > 0).astype(jnp.int32), axis=0,
                           keepdims=True)
    lse = rowmax[:, 0] + jnp.log(jnp.sum(e_all, axis=1))
    zsum_ref[0, 0] += jnp.sum(lse)

    @pl.when(i == nsteps - 1)
    def _finish():
        imp = imp_ref[:].reshape(_N_EXPERTS)
        load = load_ref[:].astype(jnp.float32).reshape(_N_EXPERTS)
        z = zsum_ref[0, 0] / jnp.float32(_N_TOKENS)
        loss_ref[0, 0] = _cv2(imp) + _cv2(load) + z


@jax.jit
def kernel(flat_tokens, gate_weight, noise_weight):
    del noise_weight
    n_tokens = flat_tokens.shape[0]
    grid = (n_tokens // _ROWS,)
    out_shape = (
        jax.ShapeDtypeStruct((n_tokens, _N_EXPERTS), jnp.float32),
        jax.ShapeDtypeStruct((n_tokens, _N_EXPERTS), jnp.float32),
        jax.ShapeDtypeStruct((1, _N_EXPERTS), jnp.float32),
        jax.ShapeDtypeStruct((1, _N_EXPERTS), jnp.int32),
        jax.ShapeDtypeStruct((1, 1), jnp.float32),
    )
    in_specs = [
        pl.BlockSpec((_ROWS, _IN_DIM), lambda i: (i, 0)),
        pl.BlockSpec((_IN_DIM, _N_EXPERTS), lambda i: (0, 0)),
    ]
    out_specs = (
        pl.BlockSpec((_ROWS, _N_EXPERTS), lambda i: (i, 0)),
        pl.BlockSpec((_ROWS, _N_EXPERTS), lambda i: (i, 0)),
        pl.BlockSpec((1, _N_EXPERTS), lambda i: (0, 0)),
        pl.BlockSpec((1, _N_EXPERTS), lambda i: (0, 0)),
        pl.BlockSpec(memory_space=pltpu.SMEM),
    )
    logits, gates, imp, load, loss = pl.pallas_call(
        _router_body,
        grid=grid,
        in_specs=in_specs,
        out_specs=out_specs,
        out_shape=out_shape,
        scratch_shapes=[pltpu.SMEM((1, 1), jnp.float32)],
    )(flat_tokens, gate_weight)
    return (gates, load.reshape(_N_EXPERTS), logits, loss[0, 0],
            imp.reshape(_N_EXPERTS))
